# baseline (speedup 1.0000x reference)
import numpy as np
import jax
import jax.numpy as jnp
from jax import lax
from jax.experimental import pallas as pl
from jax.experimental.pallas import tpu as pltpu
from jax.experimental.pallas import tpu_sc as plsc

_N, _E, _D, _H, _G, _C = 10000, 320000, 128, 128, 64, 10
_BN_SCALE = float(1.0 / np.sqrt(1.0 + 1e-5))

_NC, _NS = 2, 16
_NW = _NC * _NS
_CH = 80
_SB = 4
_NSB = _E // (_CH * _SB)
_SPT = _NSB // _NW
_SREM = _NSB - _SPT * _NW
_RPT = 632
_LASTR = _N - _RPT * (_NS - 1)

_R = 2000
_NB = _N // _R


def _agg_body(x_hbm, src_hbm, dst_hbm, zinit_hbm, out_hbm, srcv, dstv, rows_v,
              acc_sh, gs0, gs1, gs2, gs3, ss0, ss1, ss2, ss3, isem):
    c = lax.axis_index("c")
    s = lax.axis_index("s")
    w = s * _NC + c

    base = w * _SPT + jnp.minimum(w, _SREM)
    ngroups = _SPT + (w < _SREM).astype(jnp.int32)
    gsems = (gs0, gs1, gs2, gs3)
    ssems = (ss0, ss1, ss2, ss3)
    dummy = x_hbm.at[pl.ds(0, _CH), :]
    idummy = src_hbm.at[0]
    r0 = s * _RPT

    def idx_fire(q, slot):
        q = jnp.minimum(q, _NSB - 1)
        pltpu.async_copy(src_hbm.at[q], srcv.at[slot], isem)
        pltpu.async_copy(dst_hbm.at[q], dstv.at[slot], isem)

    def idx_drain(slot):
        pltpu.make_async_copy(idummy, srcv.at[slot], isem).wait()
        pltpu.make_async_copy(idummy, dstv.at[slot], isem).wait()

    def g_fire(slot, j, b):
        pltpu.async_copy(x_hbm.at[srcv.at[slot, j]], rows_v.at[b], gsems[b])

    def s_fire(slot, j, b):
        pltpu.async_copy(rows_v.at[b], acc_sh.at[dstv.at[slot, j]], ssems[b],
                         add=True)

    def g_drain(b):
        pltpu.make_async_copy(dummy, rows_v.at[b], gsems[b]).wait()

    def s_drain(b):
        pltpu.make_async_copy(dummy, rows_v.at[b], ssems[b]).wait()

    def zinit_copy():
        @pl.when(s < _NS - 1)
        def _():
            pltpu.sync_copy(zinit_hbm.at[pl.ds(r0, _RPT), :],
                            acc_sh.at[pl.ds(r0, _RPT), :])

        @pl.when(s == _NS - 1)
        def _():
            pltpu.sync_copy(zinit_hbm.at[pl.ds(_RPT * (_NS - 1), _LASTR), :],
                            acc_sh.at[pl.ds(_RPT * (_NS - 1), _LASTR), :])

    def group(g, peel):
        cur = lax.rem(g, 3)
        nxt = lax.rem(g + 1, 3)
        fut = lax.rem(g + 2, 3)
        last = g + 1 >= ngroups
        if not peel:
            s_drain(2)
        g_fire(cur, 2, 2)
        g_drain(0); s_fire(cur, 0, 0)
        if not peel:
            s_drain(3)
        idx_fire(base + g + 2, fut)
        g_fire(cur, 3, 3)
        g_drain(1); s_fire(cur, 1, 1)
        s_drain(0)
        idx_drain(nxt)

        @pl.when(jnp.logical_not(last))
        def _():
            g_fire(nxt, 0, 0)

        g_drain(2); s_fire(cur, 2, 2)
        s_drain(1)

        @pl.when(jnp.logical_not(last))
        def _():
            g_fire(nxt, 1, 1)

        g_drain(3); s_fire(cur, 3, 3)

    idx_fire(base, 0)
    idx_drain(0)
    g_fire(0, 0, 0)
    g_fire(0, 1, 1)
    idx_fire(base + 1, 1)
    zinit_copy()
    plsc.subcore_barrier()

    group(0, True)
    lax.fori_loop(1, ngroups, lambda g, car: (group(g, False), car)[1], 0)

    idx_drain(lax.rem(ngroups + 1, 3))
    s_drain(2)
    s_drain(3)

    plsc.subcore_barrier()

    @pl.when(s < _NS - 1)
    def _():
        pltpu.sync_copy(acc_sh.at[pl.ds(r0, _RPT), :],
                        out_hbm.at[c, pl.ds(r0, _RPT), :])

    @pl.when(s == _NS - 1)
    def _():
        pltpu.sync_copy(acc_sh.at[pl.ds(_RPT * (_NS - 1), _LASTR), :],
                        out_hbm.at[c, pl.ds(_RPT * (_NS - 1), _LASTR), :])


_agg_kernel_cache = {}


def _agg_run(x, src, dst, zinit):
    if "k" not in _agg_kernel_cache:
        _agg_kernel_cache["k"] = pl.kernel(
            _agg_body,
            out_type=jax.ShapeDtypeStruct((_NC, _N, _D), jnp.float32),
            mesh=plsc.VectorSubcoreMesh(core_axis_name="c",
                                        subcore_axis_name="s",
                                        num_cores=_NC, num_subcores=_NS),
            scratch_types=[
                pltpu.VMEM((3, _SB, _CH), jnp.int32),
                pltpu.VMEM((3, _SB, _CH), jnp.int32),
                pltpu.VMEM((4, _CH, _D), jnp.float32),
                pltpu.VMEM_SHARED((_N, _D), jnp.float32),
                pltpu.SemaphoreType.DMA,
                pltpu.SemaphoreType.DMA,
                pltpu.SemaphoreType.DMA,
                pltpu.SemaphoreType.DMA,
                pltpu.SemaphoreType.DMA,
                pltpu.SemaphoreType.DMA,
                pltpu.SemaphoreType.DMA,
                pltpu.SemaphoreType.DMA,
                pltpu.SemaphoreType.DMA,
            ],
        )
    return _agg_kernel_cache["k"](x, src, dst, zinit)


def _mlp2_body(x_ref, a_ref, w1_ref, b1_ref, w2_ref, b2_ref, g_ref, be_ref,
               bat_ref, h_ref, p_ref):
    i = pl.program_id(0)
    h = x_ref[...] + a_ref[0] + a_ref[1]
    t = jnp.maximum(
        jnp.dot(h, w1_ref[...], preferred_element_type=jnp.float32)
        + b1_ref[...], 0.0)
    u = (jnp.dot(t, w2_ref[...], preferred_element_type=jnp.float32)
         + b2_ref[...])
    hn = jnp.maximum(u * _BN_SCALE * g_ref[...] + be_ref[...], 0.0)
    h_ref[...] = hn
    bat = bat_ref[0, 0, :]
    oh = (bat[:, None] == lax.broadcasted_iota(jnp.int32, (_R, _G), 1)
          ).astype(jnp.float32)
    p = lax.dot_general(oh, hn, (((0,), (0,)), ((), ())),
                        preferred_element_type=jnp.float32)

    @pl.when(i == 0)
    def _():
        p_ref[...] = p

    @pl.when(i != 0)
    def _():
        p_ref[...] = p_ref[...] + p


def _mlp3_body(x_ref, a_ref, w_ref, b_ref, g_ref, be_ref, bat_ref, p_ref):
    i = pl.program_id(0)
    h = x_ref[...] + a_ref[0] + a_ref[1]
    t = jnp.maximum(
        jnp.dot(h, w_ref[...], preferred_element_type=jnp.float32)
        + b_ref[...], 0.0)
    hn = jnp.maximum(t * _BN_SCALE * g_ref[...] + be_ref[...], 0.0)
    bat = bat_ref[0, 0, :]
    oh = (bat[:, None] == lax.broadcasted_iota(jnp.int32, (_R, _G), 1)
          ).astype(jnp.float32)
    p = lax.dot_general(oh, hn, (((0,), (0,)), ((), ())),
                        preferred_element_type=jnp.float32)

    @pl.when(i == 0)
    def _():
        p_ref[...] = p

    @pl.when(i != 0)
    def _():
        p_ref[...] = p_ref[...] + p


def _readout_body(p1_ref, p2_ref, p3_ref, wa_ref, wb_ref, wc_ref, b1_ref,
                  w2_ref, b2_ref, out_ref):
    z = (jnp.dot(p1_ref[...], wa_ref[...], preferred_element_type=jnp.float32)
         + jnp.dot(p2_ref[...], wb_ref[...], preferred_element_type=jnp.float32)
         + jnp.dot(p3_ref[...], wc_ref[...], preferred_element_type=jnp.float32)
         + b1_ref[...])
    z = jnp.maximum(z, 0.0)
    out_ref[...] = (jnp.dot(z, w2_ref[...], preferred_element_type=jnp.float32)
                    + b2_ref[...])


def _full_spec(shape):
    nd = len(shape)
    return pl.BlockSpec(shape, lambda i=0, _n=nd: (0,) * _n)


def _mlp2_call(x, a, w1, b1, w2, b2, g, be, bat3, dh):
    return pl.pallas_call(
        _mlp2_body,
        grid=(_NB,),
        in_specs=[
            pl.BlockSpec((_R, _D), lambda i: (i, 0)),
            pl.BlockSpec((_NC, _R, _D), lambda i: (0, i, 0)),
            _full_spec((_D, dh)),
            _full_spec((1, dh)),
            _full_spec((dh, dh)),
            _full_spec((1, dh)),
            _full_spec((1, dh)),
            _full_spec((1, dh)),
            pl.BlockSpec((1, 1, _R), lambda i: (i, 0, 0)),
        ],
        out_specs=[
            pl.BlockSpec((_R, dh), lambda i: (i, 0)),
            pl.BlockSpec((_G, dh), lambda i: (0, 0)),
        ],
        out_shape=[
            jax.ShapeDtypeStruct((_N, dh), jnp.float32),
            jax.ShapeDtypeStruct((_G, dh), jnp.float32),
        ],
    )(x, a, w1, b1, w2, b2, g, be, bat3)


def _mlp3_call(x, a, w, b, g, be, bat3, dh):
    return pl.pallas_call(
        _mlp3_body,
        grid=(_NB,),
        in_specs=[
            pl.BlockSpec((_R, _D), lambda i: (i, 0)),
            pl.BlockSpec((_NC, _R, _D), lambda i: (0, i, 0)),
            _full_spec((_D, dh)),
            _full_spec((1, dh)),
            _full_spec((1, dh)),
            _full_spec((1, dh)),
            pl.BlockSpec((1, 1, _R), lambda i: (i, 0, 0)),
        ],
        out_specs=pl.BlockSpec((_G, dh), lambda i: (0, 0)),
        out_shape=jax.ShapeDtypeStruct((_G, dh), jnp.float32),
    )(x, a, w, b, g, be, bat3)


def _readout_call(p1, p2, p3, wa, wb, wc, b1, w2, b2):
    return pl.pallas_call(
        _readout_body,
        in_specs=[_full_spec(t.shape) for t in
                  (p1, p2, p3, wa, wb, wc, b1, w2, b2)],
        out_specs=_full_spec((_G, _C)),
        out_shape=jax.ShapeDtypeStruct((_G, _C), jnp.float32),
    )(p1, p2, p3, wa, wb, wc, b1, w2, b2)


def kernel(x, edge_index, batch, W1a, b1a, W1b, b1b, W2a, b2a, W2b, b2b, W3,
           b3, g1, be1, g2, be2, g3, be3, Wl1, bl1, Wl2, bl2):
    src = edge_index[0].reshape(_NSB, _SB, _CH)
    dst = edge_index[1].reshape(_NSB, _SB, _CH)
    bat3 = batch.reshape(_NB, 1, _R)
    zinit = jnp.zeros((_N, _D), jnp.float32)

    r = lambda v: v.reshape(1, -1)

    a = _agg_run(x, src, dst, zinit)
    h1, p1 = _mlp2_call(x, a, W1a, r(b1a), W1b, r(b1b), r(g1), r(be1), bat3,
                        _H)
    a = _agg_run(h1, src, dst, zinit)
    h2, p2 = _mlp2_call(h1, a, W2a, r(b2a), W2b, r(b2b), r(g2), r(be2), bat3,
                        _H)
    a = _agg_run(h2, src, dst, zinit)
    p3 = _mlp3_call(h2, a, W3, r(b3), r(g3), r(be3), bat3, 512)

    return _readout_call(p1, p2, p3, Wl1[:_H], Wl1[_H:2 * _H], Wl1[2 * _H:],
                         r(bl1), Wl2, r(bl2))


# speedup vs baseline: 1.1340x; 1.1340x over previous
import numpy as np
import jax
import jax.numpy as jnp
from jax import lax
from jax.experimental import pallas as pl
from jax.experimental.pallas import tpu as pltpu
from jax.experimental.pallas import tpu_sc as plsc

_N, _E, _D, _H, _G, _C = 10000, 320000, 128, 128, 64, 10
_BN_SCALE = float(1.0 / np.sqrt(1.0 + 1e-5))

_NC, _NS = 2, 16
_NW = _NC * _NS
_CH = 128
_SB = 2
_NSB = _E // (_CH * _SB)
_SPT = _NSB // _NW
_SREM = _NSB - _SPT * _NW
_NGRP = _SPT // 3
_RPT = 632
_LASTR = _N - _RPT * (_NS - 1)

_R = 2000
_NB = _N // _R


def _agg_body(x_hbm, src_hbm, dst_hbm, zinit_hbm, out_hbm, srcv, dstv, rows_v,
              acc_sh, gs0, gs1, gs2, ss0, ss1, ss2, isem):
    c = lax.axis_index("c")
    s = lax.axis_index("s")
    w = s * _NC + c

    base = w * _SPT + jnp.minimum(w, _SREM)
    total = 6 * _NGRP + 2 * (w < _SREM).astype(jnp.int32)
    gsems = (gs0, gs1, gs2)
    ssems = (ss0, ss1, ss2)
    dummy = x_hbm.at[pl.ds(0, _CH), :]
    r0 = s * _RPT

    def idx_fire(q, slot):
        q = jnp.minimum(q, _NSB - 1)
        pltpu.async_copy(src_hbm.at[q], srcv.at[slot], isem)
        pltpu.async_copy(dst_hbm.at[q], dstv.at[slot], isem)

    def idx_drain(slot):
        pltpu.make_async_copy(dummy, srcv.at[slot], isem).wait()
        pltpu.make_async_copy(dummy, dstv.at[slot], isem).wait()

    def g_fire(slot, j, b):
        pltpu.async_copy(x_hbm.at[srcv.at[slot, j]], rows_v.at[b], gsems[b])

    def s_fire(slot, j, b):
        pltpu.async_copy(rows_v.at[b], acc_sh.at[dstv.at[slot, j]], ssems[b],
                         add=True)

    def g_drain(b):
        pltpu.make_async_copy(dummy, rows_v.at[b], gsems[b]).wait()

    def s_drain(b):
        pltpu.make_async_copy(dummy, rows_v.at[b], ssems[b]).wait()

    def zinit_copy():
        @pl.when(s < _NS - 1)
        def _():
            pltpu.sync_copy(zinit_hbm.at[pl.ds(r0, _RPT), :],
                            acc_sh.at[pl.ds(r0, _RPT), :])

        @pl.when(s == _NS - 1)
        def _():
            pltpu.sync_copy(zinit_hbm.at[pl.ds(_RPT * (_NS - 1), _LASTR), :],
                            acc_sh.at[pl.ds(_RPT * (_NS - 1), _LASTR), :])

    def group(g, peel):
        c6 = 6 * g
        if not peel:
            s_drain(2)
        idx_drain(1)
        idx_fire(base + 3 * g + 2, 2)
        g_fire(1, 0, 2)
        g_drain(0); s_fire(0, 0, 0)
        s_drain(0)
        g_fire(1, 1, 0)
        g_drain(1); s_fire(0, 1, 1)
        s_drain(1)
        idx_drain(2)
        idx_fire(base + 3 * g + 3, 0)
        g_fire(2, 0, 1)
        g_drain(2); s_fire(1, 0, 2)
        s_drain(2)
        g_fire(2, 1, 2)
        g_drain(0); s_fire(1, 1, 0)
        s_drain(0)
        idx_drain(0)
        idx_fire(base + 3 * g + 4, 1)

        @pl.when(c6 + 6 < total)
        def _():
            g_fire(0, 0, 0)

        g_drain(1); s_fire(2, 0, 1)
        s_drain(1)

        @pl.when(c6 + 7 < total)
        def _():
            g_fire(0, 1, 1)

        g_drain(2); s_fire(2, 1, 2)

    idx_fire(base, 0)
    idx_drain(0)
    g_fire(0, 0, 0)
    g_fire(0, 1, 1)
    idx_fire(base + 1, 1)
    zinit_copy()
    plsc.subcore_barrier()

    group(0, True)
    lax.fori_loop(1, _NGRP, lambda g, car: (group(g, False), car)[1], 0)

    rem = total > 6 * _NGRP

    @pl.when(rem)
    def _():
        g_drain(0); s_fire(0, 0, 0)
        g_drain(1); s_fire(0, 1, 1)

    s_drain(2)

    @pl.when(rem)
    def _():
        s_drain(0)
        s_drain(1)

    idx_drain(1)

    plsc.subcore_barrier()

    @pl.when(s < _NS - 1)
    def _():
        pltpu.sync_copy(acc_sh.at[pl.ds(r0, _RPT), :],
                        out_hbm.at[c, pl.ds(r0, _RPT), :])

    @pl.when(s == _NS - 1)
    def _():
        pltpu.sync_copy(acc_sh.at[pl.ds(_RPT * (_NS - 1), _LASTR), :],
                        out_hbm.at[c, pl.ds(_RPT * (_NS - 1), _LASTR), :])


_agg_kernel_cache = {}


def _agg_run(x, src, dst, zinit):
    if "k" not in _agg_kernel_cache:
        _agg_kernel_cache["k"] = pl.kernel(
            _agg_body,
            out_type=jax.ShapeDtypeStruct((_NC, _N, _D), jnp.float32),
            mesh=plsc.VectorSubcoreMesh(core_axis_name="c",
                                        subcore_axis_name="s",
                                        num_cores=_NC, num_subcores=_NS),
            scratch_types=[
                pltpu.VMEM((3, _SB, _CH), jnp.int32),
                pltpu.VMEM((3, _SB, _CH), jnp.int32),
                pltpu.VMEM((3, _CH, _D), jnp.float32),
                pltpu.VMEM_SHARED((_N, _D), jnp.float32),
                pltpu.SemaphoreType.DMA,
                pltpu.SemaphoreType.DMA,
                pltpu.SemaphoreType.DMA,
                pltpu.SemaphoreType.DMA,
                pltpu.SemaphoreType.DMA,
                pltpu.SemaphoreType.DMA,
                pltpu.SemaphoreType.DMA,
            ],
        )
    return _agg_kernel_cache["k"](x, src, dst, zinit)


def _mlp2_body(x_ref, a_ref, w1_ref, b1_ref, w2_ref, b2_ref, g_ref, be_ref,
               bat_ref, h_ref, p_ref):
    i = pl.program_id(0)
    h = x_ref[...] + a_ref[0] + a_ref[1]
    t = jnp.maximum(
        jnp.dot(h, w1_ref[...], preferred_element_type=jnp.float32)
        + b1_ref[...], 0.0)
    u = (jnp.dot(t, w2_ref[...], preferred_element_type=jnp.float32)
         + b2_ref[...])
    hn = jnp.maximum(u * _BN_SCALE * g_ref[...] + be_ref[...], 0.0)
    h_ref[...] = hn
    bat = bat_ref[0, 0, :]
    oh = (bat[:, None] == lax.broadcasted_iota(jnp.int32, (_R, _G), 1)
          ).astype(jnp.float32)
    p = lax.dot_general(oh, hn, (((0,), (0,)), ((), ())),
                        preferred_element_type=jnp.float32)

    @pl.when(i == 0)
    def _():
        p_ref[...] = p

    @pl.when(i != 0)
    def _():
        p_ref[...] = p_ref[...] + p


def _mlp3_body(x_ref, a_ref, w_ref, b_ref, g_ref, be_ref, bat_ref, p1_ref,
               p2_ref, wa_ref, wb_ref, wc_ref, b1_ref, w2_ref, b2_ref,
               out_ref, p_acc):
    i = pl.program_id(0)
    h = x_ref[...] + a_ref[0] + a_ref[1]
    t = jnp.maximum(
        jnp.dot(h, w_ref[...], preferred_element_type=jnp.float32)
        + b_ref[...], 0.0)
    hn = jnp.maximum(t * _BN_SCALE * g_ref[...] + be_ref[...], 0.0)
    bat = bat_ref[0, 0, :]
    oh = (bat[:, None] == lax.broadcasted_iota(jnp.int32, (_R, _G), 1)
          ).astype(jnp.float32)
    p = lax.dot_general(oh, hn, (((0,), (0,)), ((), ())),
                        preferred_element_type=jnp.float32)

    @pl.when(i == 0)
    def _():
        p_acc[...] = p

    @pl.when(i != 0)
    def _():
        p_acc[...] = p_acc[...] + p

    @pl.when(i == _NB - 1)
    def _():
        z = (jnp.dot(p1_ref[...], wa_ref[...],
                     preferred_element_type=jnp.float32)
             + jnp.dot(p2_ref[...], wb_ref[...],
                       preferred_element_type=jnp.float32)
             + jnp.dot(p_acc[...], wc_ref[...],
                       preferred_element_type=jnp.float32)
             + b1_ref[...])
        z = jnp.maximum(z, 0.0)
        out_ref[...] = (jnp.dot(z, w2_ref[...],
                                preferred_element_type=jnp.float32)
                        + b2_ref[...])


def _readout_body(p1_ref, p2_ref, p3_ref, wa_ref, wb_ref, wc_ref, b1_ref,
                  w2_ref, b2_ref, out_ref):
    z = (jnp.dot(p1_ref[...], wa_ref[...], preferred_element_type=jnp.float32)
         + jnp.dot(p2_ref[...], wb_ref[...], preferred_element_type=jnp.float32)
         + jnp.dot(p3_ref[...], wc_ref[...], preferred_element_type=jnp.float32)
         + b1_ref[...])
    z = jnp.maximum(z, 0.0)
    out_ref[...] = (jnp.dot(z, w2_ref[...], preferred_element_type=jnp.float32)
                    + b2_ref[...])


def _full_spec(shape):
    nd = len(shape)
    return pl.BlockSpec(shape, lambda i=0, _n=nd: (0,) * _n)


def _mlp2_call(x, a, w1, b1, w2, b2, g, be, bat3, dh):
    return pl.pallas_call(
        _mlp2_body,
        grid=(_NB,),
        in_specs=[
            pl.BlockSpec((_R, _D), lambda i: (i, 0)),
            pl.BlockSpec((_NC, _R, _D), lambda i: (0, i, 0)),
            _full_spec((_D, dh)),
            _full_spec((1, dh)),
            _full_spec((dh, dh)),
            _full_spec((1, dh)),
            _full_spec((1, dh)),
            _full_spec((1, dh)),
            pl.BlockSpec((1, 1, _R), lambda i: (i, 0, 0)),
        ],
        out_specs=[
            pl.BlockSpec((_R, dh), lambda i: (i, 0)),
            pl.BlockSpec((_G, dh), lambda i: (0, 0)),
        ],
        out_shape=[
            jax.ShapeDtypeStruct((_N, dh), jnp.float32),
            jax.ShapeDtypeStruct((_G, dh), jnp.float32),
        ],
    )(x, a, w1, b1, w2, b2, g, be, bat3)


def _mlp3_call(x, a, w, b, g, be, bat3, p1, p2, wa, wb, wc, b1, w2, b2, dh):
    return pl.pallas_call(
        _mlp3_body,
        grid=(_NB,),
        in_specs=[
            pl.BlockSpec((_R, _D), lambda i: (i, 0)),
            pl.BlockSpec((_NC, _R, _D), lambda i: (0, i, 0)),
            _full_spec((_D, dh)),
            _full_spec((1, dh)),
            _full_spec((1, dh)),
            _full_spec((1, dh)),
            pl.BlockSpec((1, 1, _R), lambda i: (i, 0, 0)),
            _full_spec((_G, _H)),
            _full_spec((_G, _H)),
            _full_spec((_H, 1024)),
            _full_spec((_H, 1024)),
            _full_spec((dh, 1024)),
            _full_spec((1, 1024)),
            _full_spec((1024, _C)),
            _full_spec((1, _C)),
        ],
        out_specs=pl.BlockSpec((_G, _C), lambda i: (0, 0)),
        out_shape=jax.ShapeDtypeStruct((_G, _C), jnp.float32),
        scratch_shapes=[pltpu.VMEM((_G, dh), jnp.float32)],
    )(x, a, w, b, g, be, bat3, p1, p2, wa, wb, wc, b1, w2, b2)


def _readout_call(p1, p2, p3, wa, wb, wc, b1, w2, b2):
    return pl.pallas_call(
        _readout_body,
        in_specs=[_full_spec(t.shape) for t in
                  (p1, p2, p3, wa, wb, wc, b1, w2, b2)],
        out_specs=_full_spec((_G, _C)),
        out_shape=jax.ShapeDtypeStruct((_G, _C), jnp.float32),
    )(p1, p2, p3, wa, wb, wc, b1, w2, b2)


def kernel(x, edge_index, batch, W1a, b1a, W1b, b1b, W2a, b2a, W2b, b2b, W3,
           b3, g1, be1, g2, be2, g3, be3, Wl1, bl1, Wl2, bl2):
    src = edge_index[0].reshape(_NSB, _SB, _CH)
    dst = edge_index[1].reshape(_NSB, _SB, _CH)
    bat3 = batch.reshape(_NB, 1, _R)
    zinit = jnp.zeros((_N, _D), jnp.float32)

    r = lambda v: v.reshape(1, -1)

    a = _agg_run(x, src, dst, zinit)
    h1, p1 = _mlp2_call(x, a, W1a, r(b1a), W1b, r(b1b), r(g1), r(be1), bat3,
                        _H)
    a = _agg_run(h1, src, dst, zinit)
    h2, p2 = _mlp2_call(h1, a, W2a, r(b2a), W2b, r(b2b), r(g2), r(be2), bat3,
                        _H)
    a = _agg_run(h2, src, dst, zinit)
    return _mlp3_call(h2, a, W3, r(b3), r(g3), r(be3), bat3, p1, p2,
                      Wl1[:_H], Wl1[_H:2 * _H], Wl1[2 * _H:], r(bl1), Wl2,
                      r(bl2), 512)


# speedup vs baseline: 1.1439x; 1.0087x over previous
import numpy as np
import jax
import jax.numpy as jnp
from jax import lax
from jax.experimental import pallas as pl
from jax.experimental.pallas import tpu as pltpu
from jax.experimental.pallas import tpu_sc as plsc

_N, _E, _D, _H, _G, _C = 10000, 320000, 128, 128, 64, 10
_BN_SCALE = float(1.0 / np.sqrt(1.0 + 1e-5))

_NC, _NS = 2, 16
_NW = _NC * _NS
_CH = 128
_SB = 2
_NSB = _E // (_CH * _SB)
_SPT = _NSB // _NW
_SREM = _NSB - _SPT * _NW
_NGRP = _SPT // 3
_RPT = 632
_LASTR = _N - _RPT * (_NS - 1)

_R = 2000
_NB = _N // _R


def _agg_body(x_hbm, src_hbm, dst_hbm, zinit_hbm, out_hbm, srcv, dstv, rows_v,
              acc_sh, gs0, gs1, gs2, ss0, ss1, ss2, isem):
    c = lax.axis_index("c")
    s = lax.axis_index("s")
    w = s * _NC + c

    base = w * _SPT + jnp.minimum(w, _SREM)
    total = 6 * _NGRP + 2 * (w < _SREM).astype(jnp.int32)
    gsems = (gs0, gs1, gs2)
    ssems = (ss0, ss1, ss2)
    dummy = x_hbm.at[pl.ds(0, _CH), :]
    r0 = s * _RPT

    def idx_fire(q, slot):
        q = jnp.minimum(q, _NSB - 1)
        pltpu.async_copy(src_hbm.at[q], srcv.at[slot], isem)
        pltpu.async_copy(dst_hbm.at[q], dstv.at[slot], isem)

    def idx_drain(slot):
        pltpu.make_async_copy(dummy, srcv.at[slot], isem).wait()
        pltpu.make_async_copy(dummy, dstv.at[slot], isem).wait()

    def g_fire(slot, j, b):
        pltpu.async_copy(x_hbm.at[srcv.at[slot, j]], rows_v.at[b], gsems[b])

    def s_fire(slot, j, b):
        pltpu.async_copy(rows_v.at[b], acc_sh.at[dstv.at[slot, j]], ssems[b],
                         add=True)

    def g_drain(b):
        pltpu.make_async_copy(dummy, rows_v.at[b], gsems[b]).wait()

    def s_drain(b):
        pltpu.make_async_copy(dummy, rows_v.at[b], ssems[b]).wait()

    def zinit_copy():
        def init_from(ref):
            @pl.when(s < _NS - 1)
            def _():
                pltpu.sync_copy(ref.at[pl.ds(r0, _RPT), :],
                                acc_sh.at[pl.ds(r0, _RPT), :])

            @pl.when(s == _NS - 1)
            def _():
                pltpu.sync_copy(ref.at[pl.ds(_RPT * (_NS - 1), _LASTR), :],
                                acc_sh.at[pl.ds(_RPT * (_NS - 1), _LASTR), :])

        @pl.when(c == 0)
        def _():
            init_from(x_hbm)

        @pl.when(c == 1)
        def _():
            init_from(zinit_hbm)

    def group(g, peel):
        c6 = 6 * g
        if not peel:
            s_drain(2)
        idx_drain(1)
        idx_fire(base + 3 * g + 2, 2)
        g_fire(1, 0, 2)
        g_drain(0); s_fire(0, 0, 0)
        s_drain(0)
        g_fire(1, 1, 0)
        g_drain(1); s_fire(0, 1, 1)
        s_drain(1)
        idx_drain(2)
        idx_fire(base + 3 * g + 3, 0)
        g_fire(2, 0, 1)
        g_drain(2); s_fire(1, 0, 2)
        s_drain(2)
        g_fire(2, 1, 2)
        g_drain(0); s_fire(1, 1, 0)
        s_drain(0)
        idx_drain(0)
        idx_fire(base + 3 * g + 4, 1)

        @pl.when(c6 + 6 < total)
        def _():
            g_fire(0, 0, 0)

        g_drain(1); s_fire(2, 0, 1)
        s_drain(1)

        @pl.when(c6 + 7 < total)
        def _():
            g_fire(0, 1, 1)

        g_drain(2); s_fire(2, 1, 2)

    idx_fire(base, 0)
    idx_drain(0)
    g_fire(0, 0, 0)
    g_fire(0, 1, 1)
    idx_fire(base + 1, 1)
    zinit_copy()
    plsc.subcore_barrier()

    group(0, True)
    lax.fori_loop(1, _NGRP, lambda g, car: (group(g, False), car)[1], 0)

    rem = total > 6 * _NGRP

    @pl.when(rem)
    def _():
        g_drain(0); s_fire(0, 0, 0)
        g_drain(1); s_fire(0, 1, 1)

    s_drain(2)

    @pl.when(rem)
    def _():
        s_drain(0)
        s_drain(1)

    idx_drain(1)

    plsc.subcore_barrier()

    @pl.when(s < _NS - 1)
    def _():
        pltpu.sync_copy(acc_sh.at[pl.ds(r0, _RPT), :],
                        out_hbm.at[c, pl.ds(r0, _RPT), :])

    @pl.when(s == _NS - 1)
    def _():
        pltpu.sync_copy(acc_sh.at[pl.ds(_RPT * (_NS - 1), _LASTR), :],
                        out_hbm.at[c, pl.ds(_RPT * (_NS - 1), _LASTR), :])


_agg_kernel_cache = {}


def _agg_run(x, src, dst, zinit):
    if "k" not in _agg_kernel_cache:
        _agg_kernel_cache["k"] = pl.kernel(
            _agg_body,
            out_type=jax.ShapeDtypeStruct((_NC, _N, _D), jnp.float32),
            mesh=plsc.VectorSubcoreMesh(core_axis_name="c",
                                        subcore_axis_name="s",
                                        num_cores=_NC, num_subcores=_NS),
            scratch_types=[
                pltpu.VMEM((3, _SB, _CH), jnp.int32),
                pltpu.VMEM((3, _SB, _CH), jnp.int32),
                pltpu.VMEM((3, _CH, _D), jnp.float32),
                pltpu.VMEM_SHARED((_N, _D), jnp.float32),
                pltpu.SemaphoreType.DMA,
                pltpu.SemaphoreType.DMA,
                pltpu.SemaphoreType.DMA,
                pltpu.SemaphoreType.DMA,
                pltpu.SemaphoreType.DMA,
                pltpu.SemaphoreType.DMA,
                pltpu.SemaphoreType.DMA,
            ],
        )
    return _agg_kernel_cache["k"](x, src, dst, zinit)


def _mlp2_body(a_ref, w1_ref, b1_ref, w2_ref, b2_ref, g_ref, be_ref,
               bat_ref, h_ref, p_ref):
    i = pl.program_id(0)
    h = a_ref[0] + a_ref[1]
    t = jnp.maximum(
        jnp.dot(h, w1_ref[...], preferred_element_type=jnp.float32)
        + b1_ref[...], 0.0)
    u = (jnp.dot(t, w2_ref[...], preferred_element_type=jnp.float32)
         + b2_ref[...])
    hn = jnp.maximum(u * _BN_SCALE * g_ref[...] + be_ref[...], 0.0)
    h_ref[...] = hn
    bat = bat_ref[0, 0, :]
    oh = (bat[:, None] == lax.broadcasted_iota(jnp.int32, (_R, _G), 1)
          ).astype(jnp.float32)
    p = lax.dot_general(oh, hn, (((0,), (0,)), ((), ())),
                        preferred_element_type=jnp.float32)

    @pl.when(i == 0)
    def _():
        p_ref[...] = p

    @pl.when(i != 0)
    def _():
        p_ref[...] = p_ref[...] + p


def _mlp3_body(a_ref, w_ref, b_ref, g_ref, be_ref, bat_ref, p1_ref,
               p2_ref, wa_ref, wb_ref, wc_ref, b1_ref, w2_ref, b2_ref,
               out_ref, p_acc):
    i = pl.program_id(0)
    h = a_ref[0] + a_ref[1]
    t = jnp.maximum(
        jnp.dot(h, w_ref[...], preferred_element_type=jnp.float32)
        + b_ref[...], 0.0)
    hn = jnp.maximum(t * _BN_SCALE * g_ref[...] + be_ref[...], 0.0)
    bat = bat_ref[0, 0, :]
    oh = (bat[:, None] == lax.broadcasted_iota(jnp.int32, (_R, _G), 1)
          ).astype(jnp.float32)
    p = lax.dot_general(oh, hn, (((0,), (0,)), ((), ())),
                        preferred_element_type=jnp.float32)

    @pl.when(i == 0)
    def _():
        p_acc[...] = p

    @pl.when(i != 0)
    def _():
        p_acc[...] = p_acc[...] + p

    @pl.when(i == _NB - 1)
    def _():
        z = (jnp.dot(p1_ref[...], wa_ref[...],
                     preferred_element_type=jnp.float32)
             + jnp.dot(p2_ref[...], wb_ref[...],
                       preferred_element_type=jnp.float32)
             + jnp.dot(p_acc[...], wc_ref[...],
                       preferred_element_type=jnp.float32)
             + b1_ref[...])
        z = jnp.maximum(z, 0.0)
        out_ref[...] = (jnp.dot(z, w2_ref[...],
                                preferred_element_type=jnp.float32)
                        + b2_ref[...])


def _readout_body(p1_ref, p2_ref, p3_ref, wa_ref, wb_ref, wc_ref, b1_ref,
                  w2_ref, b2_ref, out_ref):
    z = (jnp.dot(p1_ref[...], wa_ref[...], preferred_element_type=jnp.float32)
         + jnp.dot(p2_ref[...], wb_ref[...], preferred_element_type=jnp.float32)
         + jnp.dot(p3_ref[...], wc_ref[...], preferred_element_type=jnp.float32)
         + b1_ref[...])
    z = jnp.maximum(z, 0.0)
    out_ref[...] = (jnp.dot(z, w2_ref[...], preferred_element_type=jnp.float32)
                    + b2_ref[...])


def _full_spec(shape):
    nd = len(shape)
    return pl.BlockSpec(shape, lambda i=0, _n=nd: (0,) * _n)


def _mlp2_call(a, w1, b1, w2, b2, g, be, bat3, dh):
    return pl.pallas_call(
        _mlp2_body,
        grid=(_NB,),
        in_specs=[
            pl.BlockSpec((_NC, _R, _D), lambda i: (0, i, 0)),
            _full_spec((_D, dh)),
            _full_spec((1, dh)),
            _full_spec((dh, dh)),
            _full_spec((1, dh)),
            _full_spec((1, dh)),
            _full_spec((1, dh)),
            pl.BlockSpec((1, 1, _R), lambda i: (i, 0, 0)),
        ],
        out_specs=[
            pl.BlockSpec((_R, dh), lambda i: (i, 0)),
            pl.BlockSpec((_G, dh), lambda i: (0, 0)),
        ],
        out_shape=[
            jax.ShapeDtypeStruct((_N, dh), jnp.float32),
            jax.ShapeDtypeStruct((_G, dh), jnp.float32),
        ],
    )(a, w1, b1, w2, b2, g, be, bat3)


def _mlp3_call(a, w, b, g, be, bat3, p1, p2, wa, wb, wc, b1, w2, b2, dh):
    return pl.pallas_call(
        _mlp3_body,
        grid=(_NB,),
        in_specs=[
            pl.BlockSpec((_NC, _R, _D), lambda i: (0, i, 0)),
            _full_spec((_D, dh)),
            _full_spec((1, dh)),
            _full_spec((1, dh)),
            _full_spec((1, dh)),
            pl.BlockSpec((1, 1, _R), lambda i: (i, 0, 0)),
            _full_spec((_G, _H)),
            _full_spec((_G, _H)),
            _full_spec((_H, 1024)),
            _full_spec((_H, 1024)),
            _full_spec((dh, 1024)),
            _full_spec((1, 1024)),
            _full_spec((1024, _C)),
            _full_spec((1, _C)),
        ],
        out_specs=pl.BlockSpec((_G, _C), lambda i: (0, 0)),
        out_shape=jax.ShapeDtypeStruct((_G, _C), jnp.float32),
        scratch_shapes=[pltpu.VMEM((_G, dh), jnp.float32)],
    )(a, w, b, g, be, bat3, p1, p2, wa, wb, wc, b1, w2, b2)


def _readout_call(p1, p2, p3, wa, wb, wc, b1, w2, b2):
    return pl.pallas_call(
        _readout_body,
        in_specs=[_full_spec(t.shape) for t in
                  (p1, p2, p3, wa, wb, wc, b1, w2, b2)],
        out_specs=_full_spec((_G, _C)),
        out_shape=jax.ShapeDtypeStruct((_G, _C), jnp.float32),
    )(p1, p2, p3, wa, wb, wc, b1, w2, b2)


def kernel(x, edge_index, batch, W1a, b1a, W1b, b1b, W2a, b2a, W2b, b2b, W3,
           b3, g1, be1, g2, be2, g3, be3, Wl1, bl1, Wl2, bl2):
    src = edge_index[0].reshape(_NSB, _SB, _CH)
    dst = edge_index[1].reshape(_NSB, _SB, _CH)
    bat3 = batch.reshape(_NB, 1, _R)
    zinit = jnp.zeros((_N, _D), jnp.float32)

    r = lambda v: v.reshape(1, -1)

    a = _agg_run(x, src, dst, zinit)
    h1, p1 = _mlp2_call(a, W1a, r(b1a), W1b, r(b1b), r(g1), r(be1), bat3, _H)
    a = _agg_run(h1, src, dst, zinit)
    h2, p2 = _mlp2_call(a, W2a, r(b2a), W2b, r(b2b), r(g2), r(be2), bat3, _H)
    a = _agg_run(h2, src, dst, zinit)
    return _mlp3_call(a, W3, r(b3), r(g3), r(be3), bat3, p1, p2,
                      Wl1[:_H], Wl1[_H:2 * _H], Wl1[2 * _H:], r(bl1), Wl2,
                      r(bl2), 512)


# speedup vs baseline: 1.1448x; 1.0009x over previous
import numpy as np
import jax
import jax.numpy as jnp
from jax import lax
from jax.experimental import pallas as pl
from jax.experimental.pallas import tpu as pltpu
from jax.experimental.pallas import tpu_sc as plsc

_N, _E, _D, _H, _G, _C = 10000, 320000, 128, 128, 64, 10
_BN_SCALE = float(1.0 / np.sqrt(1.0 + 1e-5))

_NC, _NS = 2, 16
_NW = _NC * _NS
_CH = 128
_SB = 2
_NSB = _E // (_CH * _SB)
_SPT = _NSB // _NW
_SREM = _NSB - _SPT * _NW
_NGRP = _SPT // 3
_RPT = 632
_LASTR = _N - _RPT * (_NS - 1)

_R = 2000
_NB = _N // _R


def _agg_body(x_hbm, src_hbm, dst_hbm, zinit_hbm, out_hbm, srcv, dstv, rows_v,
              acc_sh, gs0, gs1, gs2, ss0, ss1, ss2, isem):
    c = lax.axis_index("c")
    s = lax.axis_index("s")
    w = s * _NC + c

    base = w * _SPT + jnp.minimum(w, _SREM)
    total = 6 * _NGRP + 2 * (w < _SREM).astype(jnp.int32)
    gsems = (gs0, gs1, gs2)
    ssems = (ss0, ss1, ss2)
    dummy = x_hbm.at[pl.ds(0, _CH), :]
    r0 = s * _RPT

    def idx_fire(q, slot):
        q = jnp.minimum(q, _NSB - 1)
        pltpu.async_copy(src_hbm.at[q], srcv.at[slot], isem)
        pltpu.async_copy(dst_hbm.at[q], dstv.at[slot], isem)

    def idx_drain(slot):
        pltpu.make_async_copy(dummy, srcv.at[slot], isem).wait()
        pltpu.make_async_copy(dummy, dstv.at[slot], isem).wait()

    def g_fire(slot, j, b):
        pltpu.async_copy(x_hbm.at[srcv.at[slot, j]], rows_v.at[b], gsems[b])

    def s_fire(slot, j, b):
        pltpu.async_copy(rows_v.at[b], acc_sh.at[dstv.at[slot, j]], ssems[b],
                         add=True)

    def g_drain(b):
        pltpu.make_async_copy(dummy, rows_v.at[b], gsems[b]).wait()

    def s_drain(b):
        pltpu.make_async_copy(dummy, rows_v.at[b], ssems[b]).wait()

    def zinit_copy():
        def init_from(ref):
            @pl.when(s < _NS - 1)
            def _():
                pltpu.sync_copy(ref.at[pl.ds(r0, _RPT), :],
                                acc_sh.at[pl.ds(r0, _RPT), :])

            @pl.when(s == _NS - 1)
            def _():
                pltpu.sync_copy(ref.at[pl.ds(_RPT * (_NS - 1), _LASTR), :],
                                acc_sh.at[pl.ds(_RPT * (_NS - 1), _LASTR), :])

        @pl.when(c == 0)
        def _():
            init_from(x_hbm)

        @pl.when(c == 1)
        def _():
            init_from(zinit_hbm)

    def group(g, peel):
        c6 = 6 * g
        if not peel:
            s_drain(2)
        idx_drain(1)
        idx_fire(base + 3 * g + 2, 2)
        g_fire(1, 0, 2)
        g_drain(0); s_fire(0, 0, 0)
        s_drain(0)
        g_fire(1, 1, 0)
        g_drain(1); s_fire(0, 1, 1)
        s_drain(1)
        idx_drain(2)
        idx_fire(base + 3 * g + 3, 0)
        g_fire(2, 0, 1)
        g_drain(2); s_fire(1, 0, 2)
        s_drain(2)
        g_fire(2, 1, 2)
        g_drain(0); s_fire(1, 1, 0)
        s_drain(0)
        idx_drain(0)
        idx_fire(base + 3 * g + 4, 1)

        @pl.when(c6 + 6 < total)
        def _():
            g_fire(0, 0, 0)

        g_drain(1); s_fire(2, 0, 1)
        s_drain(1)

        @pl.when(c6 + 7 < total)
        def _():
            g_fire(0, 1, 1)

        g_drain(2); s_fire(2, 1, 2)

    idx_fire(base, 0)
    idx_drain(0)
    g_fire(0, 0, 0)
    g_fire(0, 1, 1)
    idx_fire(base + 1, 1)
    zinit_copy()
    plsc.subcore_barrier()

    group(0, True)
    lax.fori_loop(1, _NGRP, lambda g, car: (group(g, False), car)[1], 0)

    rem = total > 6 * _NGRP

    @pl.when(rem)
    def _():
        g_drain(0); s_fire(0, 0, 0)
        g_drain(1); s_fire(0, 1, 1)

    s_drain(2)

    @pl.when(rem)
    def _():
        s_drain(0)
        s_drain(1)

    idx_drain(1)

    plsc.subcore_barrier()

    @pl.when(s < _NS - 1)
    def _():
        pltpu.sync_copy(acc_sh.at[pl.ds(r0, _RPT), :],
                        out_hbm.at[c, pl.ds(r0, _RPT), :])

    @pl.when(s == _NS - 1)
    def _():
        pltpu.sync_copy(acc_sh.at[pl.ds(_RPT * (_NS - 1), _LASTR), :],
                        out_hbm.at[c, pl.ds(_RPT * (_NS - 1), _LASTR), :])


_agg_kernel_cache = {}


def _agg_run(x, src, dst, zinit):
    if "k" not in _agg_kernel_cache:
        _agg_kernel_cache["k"] = pl.kernel(
            _agg_body,
            out_type=jax.ShapeDtypeStruct((_NC, _N, _D), jnp.float32),
            mesh=plsc.VectorSubcoreMesh(core_axis_name="c",
                                        subcore_axis_name="s",
                                        num_cores=_NC, num_subcores=_NS),
            scratch_types=[
                pltpu.VMEM((3, _SB, _CH), jnp.int32),
                pltpu.VMEM((3, _SB, _CH), jnp.int32),
                pltpu.VMEM((3, _CH, _D), jnp.float32),
                pltpu.VMEM_SHARED((_N, _D), jnp.float32),
                pltpu.SemaphoreType.DMA,
                pltpu.SemaphoreType.DMA,
                pltpu.SemaphoreType.DMA,
                pltpu.SemaphoreType.DMA,
                pltpu.SemaphoreType.DMA,
                pltpu.SemaphoreType.DMA,
                pltpu.SemaphoreType.DMA,
            ],
        )
    return _agg_kernel_cache["k"](x, src, dst, zinit)


def _mlp2_body(a_ref, w1_ref, b1_ref, w2_ref, b2_ref, g_ref, be_ref,
               bat_ref, h_ref, p_ref):
    i = pl.program_id(0)
    h = a_ref[0] + a_ref[1]
    t = jnp.maximum(
        jnp.dot(h, w1_ref[...], preferred_element_type=jnp.float32)
        + b1_ref[...], 0.0)
    u = (jnp.dot(t, w2_ref[...], preferred_element_type=jnp.float32)
         + b2_ref[...])
    hn = jnp.maximum(u * _BN_SCALE * g_ref[...] + be_ref[...], 0.0)
    h_ref[...] = hn
    bat = bat_ref[0, 0, :]
    oh = (bat[:, None] == lax.broadcasted_iota(jnp.int32, (_R, _G), 1)
          ).astype(jnp.float32)
    p = lax.dot_general(oh, hn, (((0,), (0,)), ((), ())),
                        preferred_element_type=jnp.float32)

    @pl.when(i == 0)
    def _():
        p_ref[...] = p

    @pl.when(i != 0)
    def _():
        p_ref[...] = p_ref[...] + p


def _mlp3_body(a_ref, w_ref, b_ref, g_ref, be_ref, bat_ref, p1_ref,
               p2_ref, wa_ref, wb_ref, wc_ref, b1_ref, w2_ref, b2_ref,
               out_ref, p_acc):
    i = pl.program_id(0)
    h = a_ref[0] + a_ref[1]
    t = jnp.maximum(
        jnp.dot(h, w_ref[...], preferred_element_type=jnp.float32)
        + b_ref[...], 0.0)
    hn = jnp.maximum(t * _BN_SCALE * g_ref[...] + be_ref[...], 0.0)
    bat = bat_ref[0, 0, :]
    oh = (bat[:, None] == lax.broadcasted_iota(jnp.int32, (_R, _G), 1)
          ).astype(jnp.float32)
    p = lax.dot_general(oh, hn, (((0,), (0,)), ((), ())),
                        preferred_element_type=jnp.float32)

    @pl.when(i == 0)
    def _():
        p_acc[...] = p

    @pl.when(i != 0)
    def _():
        p_acc[...] = p_acc[...] + p

    @pl.when(i == _NB - 1)
    def _():
        z = (jnp.dot(p1_ref[...], wa_ref[...],
                     preferred_element_type=jnp.float32)
             + jnp.dot(p2_ref[...], wb_ref[...],
                       preferred_element_type=jnp.float32)
             + jnp.dot(p_acc[...], wc_ref[...],
                       preferred_element_type=jnp.float32)
             + b1_ref[...])
        z = jnp.maximum(z, 0.0)
        out_ref[...] = (jnp.dot(z, w2_ref[...],
                                preferred_element_type=jnp.float32)
                        + b2_ref[...])


def _full_spec(shape):
    nd = len(shape)
    return pl.BlockSpec(shape, lambda i=0, _n=nd: (0,) * _n)


def _mlp2_call(a, w1, b1, w2, b2, g, be, bat3, dh):
    return pl.pallas_call(
        _mlp2_body,
        grid=(_NB,),
        in_specs=[
            pl.BlockSpec((_NC, _R, _D), lambda i: (0, i, 0)),
            _full_spec((_D, dh)),
            _full_spec((1, dh)),
            _full_spec((dh, dh)),
            _full_spec((1, dh)),
            _full_spec((1, dh)),
            _full_spec((1, dh)),
            pl.BlockSpec((1, 1, _R), lambda i: (i, 0, 0)),
        ],
        out_specs=[
            pl.BlockSpec((_R, dh), lambda i: (i, 0)),
            pl.BlockSpec((_G, dh), lambda i: (0, 0)),
        ],
        out_shape=[
            jax.ShapeDtypeStruct((_N, dh), jnp.float32),
            jax.ShapeDtypeStruct((_G, dh), jnp.float32),
        ],
    )(a, w1, b1, w2, b2, g, be, bat3)


def _mlp3_call(a, w, b, g, be, bat3, p1, p2, wa, wb, wc, b1, w2, b2, dh):
    return pl.pallas_call(
        _mlp3_body,
        grid=(_NB,),
        in_specs=[
            pl.BlockSpec((_NC, _R, _D), lambda i: (0, i, 0)),
            _full_spec((_D, dh)),
            _full_spec((1, dh)),
            _full_spec((1, dh)),
            _full_spec((1, dh)),
            pl.BlockSpec((1, 1, _R), lambda i: (i, 0, 0)),
            _full_spec((_G, _H)),
            _full_spec((_G, _H)),
            _full_spec((_H, 1024)),
            _full_spec((_H, 1024)),
            _full_spec((dh, 1024)),
            _full_spec((1, 1024)),
            _full_spec((1024, _C)),
            _full_spec((1, _C)),
        ],
        out_specs=pl.BlockSpec((_G, _C), lambda i: (0, 0)),
        out_shape=jax.ShapeDtypeStruct((_G, _C), jnp.float32),
        scratch_shapes=[pltpu.VMEM((_G, dh), jnp.float32)],
    )(a, w, b, g, be, bat3, p1, p2, wa, wb, wc, b1, w2, b2)


def kernel(x, edge_index, batch, W1a, b1a, W1b, b1b, W2a, b2a, W2b, b2b, W3,
           b3, g1, be1, g2, be2, g3, be3, Wl1, bl1, Wl2, bl2):
    src = edge_index[0].reshape(_NSB, _SB, _CH)
    dst = edge_index[1].reshape(_NSB, _SB, _CH)
    bat3 = batch.reshape(_NB, 1, _R)
    zinit = jnp.zeros((_N, _D), jnp.float32)

    r = lambda v: v.reshape(1, -1)

    a = _agg_run(x, src, dst, zinit)
    h1, p1 = _mlp2_call(a, W1a, r(b1a), W1b, r(b1b), r(g1), r(be1), bat3, _H)
    a = _agg_run(h1, src, dst, zinit)
    h2, p2 = _mlp2_call(a, W2a, r(b2a), W2b, r(b2b), r(g2), r(be2), bat3, _H)
    a = _agg_run(h2, src, dst, zinit)
    return _mlp3_call(a, W3, r(b3), r(g3), r(be3), bat3, p1, p2,
                      Wl1[:_H], Wl1[_H:2 * _H], Wl1[2 * _H:], r(bl1), Wl2,
                      r(bl2), 512)
